# baseline (speedup 1.0000x reference)
import numpy as np

import jax
import jax.numpy as jnp
from jax import lax
from jax.experimental import pallas as pl
from jax.experimental.pallas import tpu as pltpu
from jax.experimental.pallas import tpu_sc as plsc

K_TOP = 64
NC, NS, L = 2, 16, 16
NW = NC * NS
B, N, F = 4, 8192, 2048
FPW = F // NW
NG = FPW // L
CHUNK = 512
NCHUNK = N // CHUNK
UNROLL = 2
CAP = 640
NBKT = 256

_MASK7F = np.int32(0x7FFFFFFF)


def _flip(xi):
    return lax.bitwise_xor(xi, lax.bitwise_and(lax.shift_right_arithmetic(xi, 31), _MASK7F))


def _bcast(x, dtype=jnp.int32):
    return lax.broadcast(lax.convert_element_type(x, dtype), (L,))


def _ivec(v):
    return _bcast(np.int32(v))


def _scan_desc(hist, base, target):
    z = _ivec(0)

    def body(i, carry):
        run, p, ca = carry
        bkt = NBKT - 1 - i
        h = hist[base + bkt]
        run2 = run + h
        newf = jnp.logical_and(run2 >= target, run < target)
        p = jnp.where(newf, _bcast(bkt), p)
        ca = jnp.where(newf, run, ca)
        return run2, p, ca
    _, p, ca = plsc.parallel_loop(0, NBKT, unroll=2, carry=(z, z, z))(body)
    return p, ca


def _kernel_body(in_hbm, out_hbm, buf0, buf1, cand, hist, outv, sem0, sem1):
    cid = lax.axis_index("c")
    sid = lax.axis_index("s")
    wid = sid * NC + cid
    lane = lax.iota(jnp.int32, L)
    ones = _ivec(1)
    zero = _ivec(0)
    z = _ivec(0)
    fb = wid * FPW

    def issue(row0, c, buf, sem):
        return pltpu.async_copy(
            in_hbm.at[pl.ds(row0 + c * CHUNK, CHUNK), pl.ds(fb, FPW)], buf, sem)

    def wait(buf, sem):
        pltpu.make_async_copy(
            in_hbm.at[pl.ds(0, CHUNK), pl.ds(fb, FPW)], buf, sem).wait()

    def run_pass(row0, process, init):
        issue(row0, 0, buf0, sem0)

        def pair(cp, carry):
            c0 = cp * 2
            wait(buf0, sem0)
            issue(row0, c0 + 1, buf1, sem1)
            carry = process(buf0, carry)

            wait(buf1, sem1)

            @pl.when(c0 + 2 < NCHUNK)
            def _():
                issue(row0, c0 + 2, buf0, sem0)
            carry = process(buf1, carry)
            return carry
        return lax.fori_loop(0, NCHUNK // 2, pair, init)

    def task(b, _):
        row0 = b * N

        def clr(i):
            hist[i] = z
        plsc.parallel_loop(0, NG * NBKT, unroll=4)(clr)

        def p1_process(buf, carry):
            def step(s):
                for u in range(UNROLL):
                    for g in range(NG):
                        v = buf[s * UNROLL + u, pl.ds(g * L, L)]
                        ks = _flip(plsc.bitcast(v, jnp.int32))
                        d0 = lax.bitwise_xor(lax.shift_right_logical(ks, 24),
                                             np.int32(128 + g * NBKT))
                        plsc.addupdate_scatter(hist, [d0, lane], ones)
            plsc.parallel_loop(0, CHUNK // UNROLL, unroll=4)(step)
            return carry
        run_pass(row0, p1_process, 0)

        p1s, ca0s = [], []
        for g in range(NG):
            p1, ca0 = _scan_desc(hist, g * NBKT, _bcast(K_TOP))
            p1s.append(p1)
            ca0s.append(ca0)

        def p2_process(buf, ptrs):
            def step(s, ptrs):
                ptrs = list(ptrs)
                for u in range(UNROLL):
                    for g in range(NG):
                        v = buf[s * UNROLL + u, pl.ds(g * L, L)]
                        ks = _flip(plsc.bitcast(v, jnp.int32))
                        d0 = lax.bitwise_xor(lax.shift_right_logical(ks, 24), np.int32(128))
                        m = jnp.logical_and(d0 >= p1s[g], ptrs[g] < CAP)
                        plsc.store_scatter(cand, [ptrs[g] + np.int32(g * CAP), lane],
                                           ks, mask=m)
                        ptrs[g] = ptrs[g] + jnp.where(m, ones, zero)
                return tuple(ptrs)
            return plsc.parallel_loop(0, CHUNK // UNROLL, unroll=4, carry=tuple(ptrs))(step)
        ncands = run_pass(row0, p2_process, (zero, zero, zero, zero))

        for g in range(NG):
            ncand = ncands[g]
            nmax = jnp.max(ncand)
            cbase = np.int32(g * CAP)
            hbase = g * NBKT

            r = _bcast(K_TOP) - ca0s[g]
            pref = lax.bitwise_xor(p1s[g], _bcast(128))
            for sh in (16, 8, 0):
                def rclr(i):
                    hist[hbase + i] = z
                plsc.parallel_loop(0, NBKT, unroll=4)(rclr)

                def rhist(i, pref=pref, sh=sh):
                    ks = cand[cbase + i]
                    valid = _bcast(i) < ncand
                    match = jnp.logical_and(
                        lax.shift_right_logical(ks, sh + 8) == pref, valid)
                    d = lax.bitwise_and(lax.shift_right_logical(ks, sh), np.int32(0xFF))
                    plsc.addupdate_scatter(hist, [d + np.int32(hbase), lane],
                                           ones, mask=match)
                plsc.parallel_loop(0, nmax, unroll=2)(rhist)
                p, ca = _scan_desc(hist, hbase, r)
                pref = lax.bitwise_or(lax.shift_left(pref, 8), p)
                r = r - ca

            ks_t = pref
            t_f = plsc.bitcast(_flip(ks_t), jnp.float32)

            def fill(i, t_f=t_f, g=g):
                outv[i, pl.ds(g * L, L)] = t_f
            plsc.parallel_loop(0, K_TOP, unroll=4)(fill)

            def coll(i, optr, g=g):
                ks = cand[cbase + i]
                valid = _bcast(i) < ncand
                m = jnp.logical_and(jnp.logical_and(ks > ks_t, valid), optr < K_TOP)
                v = plsc.bitcast(_flip(ks), jnp.float32)
                plsc.store_scatter(outv, [optr, lane + np.int32(g * L)], v, mask=m)
                return optr + jnp.where(m, ones, zero)
            plsc.parallel_loop(0, nmax, unroll=2, carry=zero)(coll)

            kk = 2
            while kk <= K_TOP:
                j = kk // 2
                while j >= 1:
                    lg = j.bit_length() - 1

                    def ce(q, j=j, lg=lg, kk=kk, g=g):
                        low = lax.bitwise_and(q, j - 1)
                        i = lax.bitwise_or(
                            lax.shift_left(lax.shift_right_logical(q, lg), lg + 1), low)
                        l2 = lax.bitwise_or(i, j)
                        a = outv[i, pl.ds(g * L, L)]
                        bb = outv[l2, pl.ds(g * L, L)]
                        mx = jnp.maximum(a, bb)
                        mn = jnp.minimum(a, bb)
                        up = _bcast(lax.bitwise_and(i, kk)) == 0
                        outv[i, pl.ds(g * L, L)] = jnp.where(up, mx, mn)
                        outv[l2, pl.ds(g * L, L)] = jnp.where(up, mn, mx)
                    plsc.parallel_loop(0, K_TOP // 2, unroll=2)(ce)
                    j //= 2
                kk *= 2

        pltpu.sync_copy(outv, out_hbm.at[pl.ds(b * K_TOP, K_TOP), pl.ds(fb, FPW)])
        return 0

    lax.fori_loop(0, B, task, 0)


@jax.jit
def _run(inputs2d):
    mesh = plsc.VectorSubcoreMesh(
        core_axis_name="c", subcore_axis_name="s", num_cores=NC, num_subcores=NS)
    f = pl.kernel(
        _kernel_body,
        out_type=jax.ShapeDtypeStruct((B * K_TOP, F), jnp.float32),
        mesh=mesh,
        compiler_params=pltpu.CompilerParams(use_tc_tiling_on_sc=False, needs_layout_passes=False),
        scratch_types=[
            pltpu.VMEM((CHUNK, FPW), jnp.float32),
            pltpu.VMEM((CHUNK, FPW), jnp.float32),
            pltpu.VMEM((NG * CAP, L), jnp.int32),
            pltpu.VMEM((NG * NBKT, L), jnp.int32),
            pltpu.VMEM((K_TOP, FPW), jnp.float32),
            pltpu.SemaphoreType.DMA,
            pltpu.SemaphoreType.DMA,
        ],
    )
    return f(inputs2d)


def kernel(inputs):
    out2d = _run(inputs.reshape(B * N, F))
    return out2d.reshape(B, K_TOP, F)


# speedup vs baseline: 1.1257x; 1.1257x over previous
import numpy as np

import jax
import jax.numpy as jnp
from jax import lax
from jax.experimental import pallas as pl
from jax.experimental.pallas import tpu as pltpu
from jax.experimental.pallas import tpu_sc as plsc

K_TOP = 64
NC, NS, L = 2, 16, 16
NW = NC * NS
B, N, F = 4, 8192, 2048
FPW = F // NW
NG = FPW // L
CHUNK = 512
NCHUNK = N // CHUNK
UNROLL = 2
CAP = 640
NBKT = 256

_MASK7F = np.int32(0x7FFFFFFF)


def _flip(xi):
    return lax.bitwise_xor(xi, lax.bitwise_and(lax.shift_right_arithmetic(xi, 31), _MASK7F))


def _bcast(x, dtype=jnp.int32):
    return lax.broadcast(lax.convert_element_type(x, dtype), (L,))


def _ivec(v):
    return _bcast(np.int32(v))


def _scan_desc(hist, base, target):
    z = _ivec(0)

    def body(i, carry):
        run, p, ca = carry
        bkt = NBKT - 1 - i
        h = hist[base + bkt]
        run2 = run + h
        newf = jnp.logical_and(run2 >= target, run < target)
        p = jnp.where(newf, _bcast(bkt), p)
        ca = jnp.where(newf, run, ca)
        return run2, p, ca
    _, p, ca = plsc.parallel_loop(0, NBKT, unroll=2, carry=(z, z, z))(body)
    return p, ca


def _kernel_body(in_hbm, out_hbm, buf0, buf1, cand, hist, outv, sem0, sem1):
    cid = lax.axis_index("c")
    sid = lax.axis_index("s")
    wid = sid * NC + cid
    lane = lax.iota(jnp.int32, L)
    ones = _ivec(1)
    zero = _ivec(0)
    z = _ivec(0)
    fb = wid * FPW

    def issue(row0, c, buf, sem):
        return pltpu.async_copy(
            in_hbm.at[pl.ds(row0 + c * CHUNK, CHUNK), pl.ds(fb, FPW)], buf, sem)

    def wait(buf, sem):
        pltpu.make_async_copy(
            in_hbm.at[pl.ds(0, CHUNK), pl.ds(fb, FPW)], buf, sem).wait()

    def run_pass(row0, process, init):
        issue(row0, 0, buf0, sem0)

        def pair(cp, carry):
            c0 = cp * 2
            wait(buf0, sem0)
            issue(row0, c0 + 1, buf1, sem1)
            carry = process(buf0, carry)

            wait(buf1, sem1)

            @pl.when(c0 + 2 < NCHUNK)
            def _():
                issue(row0, c0 + 2, buf0, sem0)
            carry = process(buf1, carry)
            return carry
        return lax.fori_loop(0, NCHUNK // 2, pair, init)

    def task(b, _):
        row0 = b * N

        def clr(i):
            hist[i] = z
        plsc.parallel_loop(0, NG * NBKT, unroll=4)(clr)

        def p1_process(buf, carry):
            def step(s):
                for u in range(UNROLL):
                    for g in range(NG):
                        v = buf[s * UNROLL + u, pl.ds(g * L, L)]
                        ks = _flip(plsc.bitcast(v, jnp.int32))
                        d0 = lax.bitwise_xor(lax.shift_right_logical(ks, 24),
                                             np.int32(128 + g * NBKT))
                        plsc.addupdate_scatter(hist, [d0, lane], ones)
            plsc.parallel_loop(0, CHUNK // UNROLL, unroll=4)(step)
            return carry
        run_pass(row0, p1_process, 0)

        p1s, ca0s = [], []
        for g in range(NG):
            p1, ca0 = _scan_desc(hist, g * NBKT, _bcast(K_TOP))
            p1s.append(p1)
            ca0s.append(ca0)

        def p2_process(buf, ptrs):
            def step(s, ptrs):
                ptrs = list(ptrs)
                for u in range(UNROLL):
                    for g in range(NG):
                        v = buf[s * UNROLL + u, pl.ds(g * L, L)]
                        ks = _flip(plsc.bitcast(v, jnp.int32))
                        d0 = lax.bitwise_xor(lax.shift_right_logical(ks, 24), np.int32(128))
                        m = jnp.logical_and(d0 >= p1s[g], ptrs[g] < CAP)
                        plsc.store_scatter(cand, [ptrs[g] + np.int32(g * CAP), lane],
                                           ks, mask=m)
                        ptrs[g] = ptrs[g] + jnp.where(m, ones, zero)
                return tuple(ptrs)
            return plsc.parallel_loop(0, CHUNK // UNROLL, unroll=2, carry=tuple(ptrs))(step)
        ncands = run_pass(row0, p2_process, (zero, zero, zero, zero))

        for g in range(NG):
            ncand = ncands[g]
            nmax = jnp.max(ncand)
            cbase = np.int32(g * CAP)
            hbase = g * NBKT

            r = _bcast(K_TOP) - ca0s[g]
            pref = lax.bitwise_xor(p1s[g], _bcast(128))
            for sh in (16, 8, 0):
                def rclr(i):
                    hist[hbase + i] = z
                plsc.parallel_loop(0, NBKT, unroll=4)(rclr)

                def rhist(i, pref=pref, sh=sh):
                    ks = cand[cbase + i]
                    valid = _bcast(i) < ncand
                    match = jnp.logical_and(
                        lax.shift_right_logical(ks, sh + 8) == pref, valid)
                    d = lax.bitwise_and(lax.shift_right_logical(ks, sh), np.int32(0xFF))
                    plsc.addupdate_scatter(hist, [d + np.int32(hbase), lane],
                                           ones, mask=match)
                plsc.parallel_loop(0, nmax, unroll=2)(rhist)
                p, ca = _scan_desc(hist, hbase, r)
                pref = lax.bitwise_or(lax.shift_left(pref, 8), p)
                r = r - ca

            ks_t = pref
            t_f = plsc.bitcast(_flip(ks_t), jnp.float32)

            def fill(i, t_f=t_f, g=g):
                outv[i, pl.ds(g * L, L)] = t_f
            plsc.parallel_loop(0, K_TOP, unroll=4)(fill)

            def coll(i, optr, g=g):
                ks = cand[cbase + i]
                valid = _bcast(i) < ncand
                m = jnp.logical_and(jnp.logical_and(ks > ks_t, valid), optr < K_TOP)
                v = plsc.bitcast(_flip(ks), jnp.float32)
                plsc.store_scatter(outv, [optr, lane + np.int32(g * L)], v, mask=m)
                return optr + jnp.where(m, ones, zero)
            plsc.parallel_loop(0, nmax, unroll=2, carry=zero)(coll)

            kk = 2
            while kk <= K_TOP:
                j = kk // 2
                while j >= 1:
                    lg = j.bit_length() - 1

                    def ce(q, j=j, lg=lg, kk=kk, g=g):
                        low = lax.bitwise_and(q, j - 1)
                        i = lax.bitwise_or(
                            lax.shift_left(lax.shift_right_logical(q, lg), lg + 1), low)
                        l2 = lax.bitwise_or(i, j)
                        a = outv[i, pl.ds(g * L, L)]
                        bb = outv[l2, pl.ds(g * L, L)]
                        mx = jnp.maximum(a, bb)
                        mn = jnp.minimum(a, bb)
                        up = _bcast(lax.bitwise_and(i, kk)) == 0
                        outv[i, pl.ds(g * L, L)] = jnp.where(up, mx, mn)
                        outv[l2, pl.ds(g * L, L)] = jnp.where(up, mn, mx)
                    plsc.parallel_loop(0, K_TOP // 2, unroll=2)(ce)
                    j //= 2
                kk *= 2

        pltpu.sync_copy(outv, out_hbm.at[pl.ds(b * K_TOP, K_TOP), pl.ds(fb, FPW)])
        return 0

    lax.fori_loop(0, B, task, 0)


@jax.jit
def _run(inputs2d):
    mesh = plsc.VectorSubcoreMesh(
        core_axis_name="c", subcore_axis_name="s", num_cores=NC, num_subcores=NS)
    f = pl.kernel(
        _kernel_body,
        out_type=jax.ShapeDtypeStruct((B * K_TOP, F), jnp.float32),
        mesh=mesh,
        compiler_params=pltpu.CompilerParams(use_tc_tiling_on_sc=False, needs_layout_passes=False),
        scratch_types=[
            pltpu.VMEM((CHUNK, FPW), jnp.float32),
            pltpu.VMEM((CHUNK, FPW), jnp.float32),
            pltpu.VMEM((NG * CAP, L), jnp.int32),
            pltpu.VMEM((NG * NBKT, L), jnp.int32),
            pltpu.VMEM((K_TOP, FPW), jnp.float32),
            pltpu.SemaphoreType.DMA,
            pltpu.SemaphoreType.DMA,
        ],
    )
    return f(inputs2d)


def kernel(inputs):
    out2d = _run(inputs.reshape(B * N, F))
    return out2d.reshape(B, K_TOP, F)
